# baseline (speedup 1.0000x reference)
import functools

import jax
import jax.numpy as jnp
from jax.experimental import pallas as pl
from jax.experimental.pallas import tpu as pltpu


def _gcn_body(dlf_ref, qm_ref, l_ref, a_ref, v_ref, spk_ref, w1_ref, b1_ref,
              cw_ref, cb_ref, out_ref, *, n_dia, dia, num_k):
    total = n_dia * dia
    d = l_ref.shape[1]
    f32 = jnp.float32

    scale = jnp.sum(dlf_ref[0, :]) / f32(dlf_ref.shape[1] * dia)

    spk_add = jnp.dot(qm_ref[...], spk_ref[...], preferred_element_type=f32)
    xl = (l_ref[...] + spk_add) * scale
    xa = a_ref[...] * scale
    xv = v_ref[...] * scale

    w1t = w1_ref[...].T
    b1 = b1_ref[...]
    hl = jnp.dot(xl, w1t, preferred_element_type=f32) + b1
    ha = jnp.dot(xa, w1t, preferred_element_type=f32) + b1
    hv = jnp.dot(xv, w1t, preferred_element_type=f32) + b1

    gl, ga, gv = hl, ha, hv
    inv = f32(1.0 / (dia + 2))
    for k in range(num_k):
        wkt = cw_ref[k].T * inv
        bk = cb_ref[k:k + 1, :]
        tot = gl + ga + gv
        outs = []
        for g in (gl, ga, gv):
            s = jnp.sum(g.reshape(n_dia, dia, d), axis=1, keepdims=True)
            sb = jnp.broadcast_to(s, (n_dia, dia, d)).reshape(total, d)
            agg = sb + (tot - g)
            outs.append(g + jnp.dot(agg, wkt, preferred_element_type=f32) + bk)
        gl, ga, gv = outs

    out_ref[:, 0 * d:1 * d] = xl
    out_ref[:, 1 * d:2 * d] = hl
    out_ref[:, 2 * d:3 * d] = gl
    out_ref[:, 3 * d:4 * d] = xa
    out_ref[:, 4 * d:5 * d] = ha
    out_ref[:, 5 * d:6 * d] = ga
    out_ref[:, 6 * d:7 * d] = xv
    out_ref[:, 7 * d:8 * d] = hv
    out_ref[:, 8 * d:9 * d] = gv


def kernel(a, v, l, qmask, dia_len, epoch, spk_emb, fc1_w, fc1_b, conv_w,
           conv_b):
    del epoch
    total, d = a.shape
    n_dia = dia_len.shape[0]
    dia = qmask.shape[0]
    num_k = conv_w.shape[0]
    nspk = qmask.shape[2]

    group = 10
    while n_dia % group:
        group -= 1
    rows = group * dia
    grid = (n_dia // group,)

    qm = jnp.transpose(qmask, (1, 0, 2)).reshape(total, nspk)
    dlf = dia_len.astype(jnp.float32).reshape(1, n_dia)
    b1 = fc1_b.reshape(1, -1)

    body = functools.partial(_gcn_body, n_dia=group, dia=dia, num_k=num_k)
    out = pl.pallas_call(
        body,
        grid=grid,
        in_specs=[
            pl.BlockSpec((1, n_dia), lambda i: (0, 0)),
            pl.BlockSpec((rows, nspk), lambda i: (i, 0)),
            pl.BlockSpec((rows, d), lambda i: (i, 0)),
            pl.BlockSpec((rows, d), lambda i: (i, 0)),
            pl.BlockSpec((rows, d), lambda i: (i, 0)),
            pl.BlockSpec((spk_emb.shape[0], d), lambda i: (0, 0)),
            pl.BlockSpec((d, d), lambda i: (0, 0)),
            pl.BlockSpec((1, d), lambda i: (0, 0)),
            pl.BlockSpec((num_k, d, d), lambda i: (0, 0, 0)),
            pl.BlockSpec((num_k, d), lambda i: (0, 0)),
        ],
        out_specs=pl.BlockSpec((rows, 9 * d), lambda i: (i, 0)),
        out_shape=jax.ShapeDtypeStruct((total, 9 * d), jnp.float32),
        compiler_params=pltpu.CompilerParams(
            dimension_semantics=("parallel",)),
    )(dlf, qm, l, a, v, spk_emb, fc1_w, b1, conv_w, conv_b)
    return out


# speedup vs baseline: 1.0748x; 1.0748x over previous
import functools

import jax
import jax.numpy as jnp
from jax.experimental import pallas as pl
from jax.experimental.pallas import tpu as pltpu


def _gcn_body(dlf_ref, qm_ref, l_ref, a_ref, v_ref, spk_ref, w1_ref, b1_ref,
              cw_ref, cb_ref, out_ref, stage_ref, sems, *, n_dia, dia, num_k):
    total = n_dia * dia
    d = l_ref.shape[1]
    f32 = jnp.float32

    def put(col, val):
        stage_ref[:, col * d:(col + 1) * d] = val
        cp = pltpu.make_async_copy(
            stage_ref.at[:, col * d:(col + 1) * d],
            out_ref.at[:, col * d:(col + 1) * d],
            sems.at[col])
        cp.start()

    def wait(col):
        pltpu.make_async_copy(
            stage_ref.at[:, col * d:(col + 1) * d],
            out_ref.at[:, col * d:(col + 1) * d],
            sems.at[col]).wait()

    scale = jnp.sum(dlf_ref[0, :]) / f32(dlf_ref.shape[1] * dia)

    spk_add = jnp.dot(qm_ref[...], spk_ref[...], preferred_element_type=f32)
    xl = (l_ref[...] + spk_add) * scale
    xa = a_ref[...] * scale
    xv = v_ref[...] * scale
    put(0, xl)
    put(3, xa)
    put(6, xv)

    w1t = w1_ref[...].T
    b1 = b1_ref[...]
    hl = jnp.dot(xl, w1t, preferred_element_type=f32) + b1
    ha = jnp.dot(xa, w1t, preferred_element_type=f32) + b1
    hv = jnp.dot(xv, w1t, preferred_element_type=f32) + b1
    put(1, hl)
    put(4, ha)
    put(7, hv)

    gl, ga, gv = hl, ha, hv
    inv = f32(1.0 / (dia + 2))
    for k in range(num_k):
        wkt = cw_ref[k].T * inv
        bk = cb_ref[k:k + 1, :]
        outs = []
        for g, o1, o2 in ((gl, ga, gv), (ga, gl, gv), (gv, gl, ga)):
            s = jnp.sum(g.reshape(n_dia, dia, d), axis=1, keepdims=True)
            sb = jnp.broadcast_to(s, (n_dia, dia, d)).reshape(total, d)
            agg = sb + (o1 + o2)
            outs.append(g + jnp.dot(agg, wkt, preferred_element_type=f32) + bk)
        gl, ga, gv = outs

    put(2, gl)
    put(5, ga)
    put(8, gv)
    for col in range(9):
        wait(col)


def kernel(a, v, l, qmask, dia_len, epoch, spk_emb, fc1_w, fc1_b, conv_w,
           conv_b):
    del epoch
    total, d = a.shape
    n_dia = dia_len.shape[0]
    dia = qmask.shape[0]
    num_k = conv_w.shape[0]
    nspk = qmask.shape[2]

    qm = jnp.transpose(qmask, (1, 0, 2)).reshape(total, nspk)
    dlf = dia_len.astype(jnp.float32).reshape(1, n_dia)
    b1 = fc1_b.reshape(1, -1)

    body = functools.partial(_gcn_body, n_dia=n_dia, dia=dia, num_k=num_k)
    out = pl.pallas_call(
        body,
        in_specs=[
            pl.BlockSpec((1, n_dia), lambda: (0, 0)),
            pl.BlockSpec((total, nspk), lambda: (0, 0)),
            pl.BlockSpec((total, d), lambda: (0, 0)),
            pl.BlockSpec((total, d), lambda: (0, 0)),
            pl.BlockSpec((total, d), lambda: (0, 0)),
            pl.BlockSpec((spk_emb.shape[0], d), lambda: (0, 0)),
            pl.BlockSpec((d, d), lambda: (0, 0)),
            pl.BlockSpec((1, d), lambda: (0, 0)),
            pl.BlockSpec((num_k, d, d), lambda: (0, 0, 0)),
            pl.BlockSpec((num_k, d), lambda: (0, 0)),
        ],
        out_specs=pl.BlockSpec(memory_space=pltpu.MemorySpace.HBM),
        out_shape=jax.ShapeDtypeStruct((total, 9 * d), jnp.float32),
        scratch_shapes=[
            pltpu.VMEM((total, 9 * d), jnp.float32),
            pltpu.SemaphoreType.DMA((9,)),
        ],
    )(dlf, qm, l, a, v, spk_emb, fc1_w, b1, conv_w, conv_b)
    return out
